# Initial kernel scaffold; baseline (speedup 1.0000x reference)
import jax
import jax.numpy as jnp
from jax.experimental import pallas as pl


def kernel(que, keys, index):
    raise NotImplementedError("write your pallas kernel here")


# speedup vs baseline: 1.4734x; 1.4734x over previous
import functools

import jax
import jax.numpy as jnp
from jax import lax
from jax.experimental import pallas as pl
from jax.experimental.pallas import tpu as pltpu
from jax.experimental.pallas import tpu_sc as plsc

_MOM = 0.9


def _sc_mask_builder(n_pad, n_idx):
    mesh = plsc.VectorSubcoreMesh(core_axis_name="c", subcore_axis_name="s")
    per_tile = n_pad // 16
    rows_per_w = n_idx // 32 // 128

    @functools.partial(
        pl.kernel,
        out_type=(
            jax.ShapeDtypeStruct((n_pad,), jnp.float32),
            jax.ShapeDtypeStruct((n_pad,), jnp.float32),
        ),
        mesh=mesh,
        scratch_types=[
            pltpu.VMEM_SHARED((n_pad,), jnp.float32),
            pltpu.VMEM((per_tile,), jnp.float32),
            pltpu.VMEM((rows_per_w, 128), jnp.int32),
            pltpu.VMEM((128,), jnp.float32),
        ],
    )
    def sc_kernel(idx_hbm, mask0_hbm, mask1_hbm, shared_mask, zbuf, idx_v,
                  ones_v):
        c = lax.axis_index("c")
        s = lax.axis_index("s")

        def _fill_zero(i, carry):
            zbuf[pl.ds(i * 16, 16)] = jnp.zeros((16,), jnp.float32)
            return carry

        lax.fori_loop(0, per_tile // 16, _fill_zero, 0)
        for i in range(8):
            ones_v[pl.ds(i * 16, 16)] = jnp.ones((16,), jnp.float32)

        sl = pl.ds(s * per_tile, per_tile)
        pltpu.sync_copy(zbuf, shared_mask.at[sl])
        plsc.subcore_barrier()

        w = s * 2 + c
        pltpu.sync_copy(idx_hbm.at[w], idx_v)
        for j in range(rows_per_w):
            pltpu.sync_copy(ones_v, shared_mask.at[idx_v.at[j]], add=True)
        plsc.subcore_barrier()

        @pl.when(c == 0)
        def _():
            pltpu.sync_copy(shared_mask.at[sl], mask0_hbm.at[sl])

        @pl.when(c == 1)
        def _():
            pltpu.sync_copy(shared_mask.at[sl], mask1_hbm.at[sl])

    return sc_kernel


def _blend_body(que_ref, keys_ref, m0_ref, m1_ref, out_ref):
    q = que_ref[...]
    flagged = m0_ref[...] + m1_ref[...]
    upd = jnp.float32(1.0 - _MOM) * keys_ref[...] + q * jnp.float32(_MOM)
    out_ref[...] = jnp.where(flagged > 0.0, upd, q)


def kernel(que, keys, index):
    n, d = que.shape
    b = index.shape[0]

    n_pad = ((n + 2047) // 2048) * 2048
    idx3 = index.astype(jnp.int32).reshape(32, b // 32 // 128, 128)
    mask0, mask1 = _sc_mask_builder(n_pad, b)(idx3)
    m0 = mask0.reshape(n_pad, 1)
    m1 = mask1.reshape(n_pad, 1)
    keys2 = keys.reshape(1, d)

    block_rows = 2000
    grid = (n // block_rows,)
    return pl.pallas_call(
        _blend_body,
        grid=grid,
        in_specs=[
            pl.BlockSpec((block_rows, d), lambda i: (i, 0)),
            pl.BlockSpec((1, d), lambda i: (0, 0)),
            pl.BlockSpec((block_rows, 1), lambda i: (i, 0)),
            pl.BlockSpec((block_rows, 1), lambda i: (i, 0)),
        ],
        out_specs=pl.BlockSpec((block_rows, d), lambda i: (i, 0)),
        out_shape=jax.ShapeDtypeStruct((n, d), jnp.float32),
    )(que, keys2, m0, m1)


# speedup vs baseline: 1.5700x; 1.0656x over previous
import functools

import jax
import jax.numpy as jnp
from jax import lax
from jax.experimental import pallas as pl
from jax.experimental.pallas import tpu as pltpu
from jax.experimental.pallas import tpu_sc as plsc

_MOM = 0.9


def _sc_mask_builder(n_pad, n_idx):
    mesh = plsc.VectorSubcoreMesh(core_axis_name="c", subcore_axis_name="s")
    per_tile = n_pad // 16
    rows_per_w = n_idx // 32 // 128

    @functools.partial(
        pl.kernel,
        out_type=(
            jax.ShapeDtypeStruct((n_pad,), jnp.float32),
            jax.ShapeDtypeStruct((n_pad,), jnp.float32),
        ),
        mesh=mesh,
        scratch_types=[
            pltpu.VMEM_SHARED((n_pad,), jnp.float32),
            pltpu.VMEM((per_tile,), jnp.float32),
            pltpu.VMEM((rows_per_w, 128), jnp.int32),
            pltpu.VMEM((128,), jnp.float32),
        ],
    )
    def sc_kernel(idx_hbm, mask0_hbm, mask1_hbm, shared_mask, zbuf, idx_v,
                  ones_v):
        c = lax.axis_index("c")
        s = lax.axis_index("s")

        def _fill_zero(i, carry):
            zbuf[pl.ds(i * 16, 16)] = jnp.zeros((16,), jnp.float32)
            return carry

        lax.fori_loop(0, per_tile // 16, _fill_zero, 0)
        for i in range(8):
            ones_v[pl.ds(i * 16, 16)] = jnp.ones((16,), jnp.float32)

        sl = pl.ds(s * per_tile, per_tile)
        pltpu.sync_copy(zbuf, shared_mask.at[sl])
        plsc.subcore_barrier()

        w = s * 2 + c
        pltpu.sync_copy(idx_hbm.at[w], idx_v)
        for j in range(rows_per_w):
            pltpu.sync_copy(ones_v, shared_mask.at[idx_v.at[j]], add=True)
        plsc.subcore_barrier()

        @pl.when(c == 0)
        def _():
            pltpu.sync_copy(shared_mask.at[sl], mask0_hbm.at[sl])

        @pl.when(c == 1)
        def _():
            pltpu.sync_copy(shared_mask.at[sl], mask1_hbm.at[sl])

    return sc_kernel


def _blend_body(que_ref, keys_ref, m0_ref, m1_ref, out_ref):
    q = que_ref[...]
    flagged = m0_ref[...] + m1_ref[...]
    upd = jnp.float32(1.0 - _MOM) * keys_ref[...] + q * jnp.float32(_MOM)
    out_ref[...] = jnp.where(flagged > 0.0, upd, q)


def kernel(que, keys, index):
    n, d = que.shape
    b = index.shape[0]

    n_pad = ((n + 2047) // 2048) * 2048
    idx3 = index.astype(jnp.int32).reshape(32, b // 32 // 128, 128)
    mask0, mask1 = _sc_mask_builder(n_pad, b)(idx3)
    m0 = mask0.reshape(n_pad, 1)
    m1 = mask1.reshape(n_pad, 1)
    keys2 = keys.reshape(1, d)

    block_rows = 4000
    grid = (n // block_rows,)
    return pl.pallas_call(
        _blend_body,
        grid=grid,
        in_specs=[
            pl.BlockSpec((block_rows, d), lambda i: (i, 0)),
            pl.BlockSpec((1, d), lambda i: (0, 0)),
            pl.BlockSpec((block_rows, 1), lambda i: (i, 0)),
            pl.BlockSpec((block_rows, 1), lambda i: (i, 0)),
        ],
        out_specs=pl.BlockSpec((block_rows, d), lambda i: (i, 0)),
        out_shape=jax.ShapeDtypeStruct((n, d), jnp.float32),
    )(que, keys2, m0, m1)


# speedup vs baseline: 2.7604x; 1.7582x over previous
import functools

import jax
import jax.numpy as jnp
from jax import lax
from jax.experimental import pallas as pl
from jax.experimental.pallas import tpu as pltpu
from jax.experimental.pallas import tpu_sc as plsc

_MOM = 0.9


def _sc_mask_builder(n_pad, n_idx):
    mesh = plsc.VectorSubcoreMesh(core_axis_name="c", subcore_axis_name="s")
    per_tile = n_pad // 16
    rows_per_w = n_idx // 32 // 128

    @functools.partial(
        pl.kernel,
        out_type=(
            jax.ShapeDtypeStruct((n_pad,), jnp.float32),
            jax.ShapeDtypeStruct((n_pad,), jnp.float32),
        ),
        mesh=mesh,
        scratch_types=[
            pltpu.VMEM_SHARED((n_pad,), jnp.float32),
            pltpu.VMEM((per_tile,), jnp.float32),
            pltpu.VMEM((rows_per_w, 128), jnp.int32),
            pltpu.VMEM((128,), jnp.float32),
        ],
    )
    def sc_kernel(idx_hbm, mask0_hbm, mask1_hbm, shared_mask, zbuf, idx_v,
                  ones_v):
        c = lax.axis_index("c")
        s = lax.axis_index("s")

        def _fill_zero(i, carry):
            zbuf[pl.ds(i * 16, 16)] = jnp.zeros((16,), jnp.float32)
            return carry

        lax.fori_loop(0, per_tile // 16, _fill_zero, 0)
        for i in range(8):
            ones_v[pl.ds(i * 16, 16)] = jnp.ones((16,), jnp.float32)

        sl = pl.ds(s * per_tile, per_tile)
        pltpu.sync_copy(zbuf, shared_mask.at[sl])
        plsc.subcore_barrier()

        w = s * 2 + c
        pltpu.sync_copy(idx_hbm.at[w], idx_v)
        for j in range(rows_per_w):
            pltpu.sync_copy(ones_v, shared_mask.at[idx_v.at[j]], add=True)
        plsc.subcore_barrier()

        @pl.when(c == 0)
        def _():
            pltpu.sync_copy(shared_mask.at[sl], mask0_hbm.at[sl])

        @pl.when(c == 1)
        def _():
            pltpu.sync_copy(shared_mask.at[sl], mask1_hbm.at[sl])

    return sc_kernel


def _blend_body(que_ref, keys_ref, m0_ref, m1_ref, out_ref):
    q = que_ref[...]
    flagged = jnp.transpose(m0_ref[0] + m1_ref[0], (1, 0))
    upd = jnp.float32(1.0 - _MOM) * keys_ref[...] + q * jnp.float32(_MOM)
    out_ref[...] = jnp.where(flagged > 0.0, upd, q)


def kernel(que, keys, index):
    n, d = que.shape
    b = index.shape[0]

    n_pad = ((n + 2047) // 2048) * 2048
    idx3 = index.astype(jnp.int32).reshape(32, b // 32 // 128, 128)
    mask0, mask1 = _sc_mask_builder(n_pad, b)(idx3)
    block_rows = 4000
    m0 = mask0[:n].reshape(n // block_rows, 1, block_rows)
    m1 = mask1[:n].reshape(n // block_rows, 1, block_rows)
    keys2 = keys.reshape(1, d)
    grid = (n // block_rows,)
    return pl.pallas_call(
        _blend_body,
        grid=grid,
        in_specs=[
            pl.BlockSpec((block_rows, d), lambda i: (i, 0)),
            pl.BlockSpec((1, d), lambda i: (0, 0)),
            pl.BlockSpec((1, 1, block_rows), lambda i: (i, 0, 0)),
            pl.BlockSpec((1, 1, block_rows), lambda i: (i, 0, 0)),
        ],
        out_specs=pl.BlockSpec((block_rows, d), lambda i: (i, 0)),
        out_shape=jax.ShapeDtypeStruct((n, d), jnp.float32),
    )(que, keys2, m0, m1)
